# Initial kernel scaffold; baseline (speedup 1.0000x reference)
import functools

import jax
import jax.numpy as jnp
from jax import lax
from jax.experimental import pallas as pl
from jax.experimental.pallas import tpu as pltpu
from jax.experimental.pallas import tpu_sc as plsc

_NUM_CLASSES = 1000000
_EMBED_DIM = 64
_BATCH = 16384
_HIST = 50
_TOTAL = _BATCH * _HIST

_info = plsc.get_sparse_core_info()
_NC = _info.num_cores
_NS = _info.num_subcores
_NW = _NC * _NS
_B_PER_W = _TOTAL // _NW
_CHUNK = 128
_NCHUNKS = _B_PER_W // _CHUNK

_mesh = plsc.VectorSubcoreMesh(core_axis_name="c", subcore_axis_name="s")


@functools.partial(
    pl.kernel,
    out_type=jax.ShapeDtypeStruct((_TOTAL, _EMBED_DIM), jnp.float32),
    mesh=_mesh,
    scratch_types=[
        pltpu.VMEM((_NCHUNKS, _CHUNK), jnp.int32),
        pltpu.VMEM((_CHUNK, _EMBED_DIM), jnp.float32),
        pltpu.SemaphoreType.DMA,
    ],
)
def _emb_lookup(idx_hbm, table_hbm, out_hbm, idx_v, rows_v, gsem):
    wid = lax.axis_index("s") * _NC + lax.axis_index("c")
    base = wid * _B_PER_W
    pltpu.sync_copy(idx_hbm.at[wid], idx_v)

    @pl.loop(0, _NCHUNKS)
    def _chunk_loop(i):
        pltpu.async_copy(table_hbm.at[idx_v.at[i]], rows_v, gsem).wait()
        pltpu.sync_copy(rows_v, out_hbm.at[pl.ds(base + i * _CHUNK, _CHUNK)])


def kernel(x, table):
    idx = x.reshape(_NW, _NCHUNKS, _CHUNK).astype(jnp.int32)
    out = _emb_lookup(idx, table)
    return out.reshape(_BATCH, _HIST, _EMBED_DIM)


# speedup vs baseline: 1.6949x; 1.6949x over previous
import functools

import jax
import jax.numpy as jnp
from jax import lax
from jax.experimental import pallas as pl
from jax.experimental.pallas import tpu as pltpu
from jax.experimental.pallas import tpu_sc as plsc

_NUM_CLASSES = 1000000
_EMBED_DIM = 64
_BATCH = 16384
_HIST = 50
_TOTAL = _BATCH * _HIST

_info = plsc.get_sparse_core_info()
_NC = _info.num_cores
_NS = _info.num_subcores
_NW = _NC * _NS
_B_PER_W = _TOTAL // _NW
_CHUNK = 128
_NCHUNKS = _B_PER_W // _CHUNK

_mesh = plsc.VectorSubcoreMesh(core_axis_name="c", subcore_axis_name="s")


@functools.partial(
    pl.kernel,
    out_type=jax.ShapeDtypeStruct((_TOTAL, _EMBED_DIM), jnp.float32),
    mesh=_mesh,
    scratch_types=[
        pltpu.VMEM((_NCHUNKS, _CHUNK), jnp.int32),
        pltpu.VMEM((_CHUNK, _EMBED_DIM), jnp.float32),
        pltpu.SemaphoreType.DMA,
    ],
    compiler_params=pltpu.CompilerParams(use_tc_tiling_on_sc=False),
)
def _emb_lookup(idx_hbm, table_hbm, out_hbm, idx_v, rows_v, gsem):
    wid = lax.axis_index("s") * _NC + lax.axis_index("c")
    base = wid * _B_PER_W
    pltpu.sync_copy(idx_hbm.at[wid], idx_v)

    @pl.loop(0, _NCHUNKS)
    def _chunk_loop(i):
        pltpu.async_copy(table_hbm.at[idx_v.at[i]], rows_v, gsem).wait()
        pltpu.sync_copy(rows_v, out_hbm.at[pl.ds(base + i * _CHUNK, _CHUNK)])


def kernel(x, table):
    idx = x.reshape(_NW, _NCHUNKS, _CHUNK).astype(jnp.int32)
    out = _emb_lookup(idx, table)
    return out.reshape(_BATCH, _HIST, _EMBED_DIM)


# speedup vs baseline: 1.8741x; 1.1057x over previous
import functools

import jax
import jax.numpy as jnp
from jax import lax
from jax.experimental import pallas as pl
from jax.experimental.pallas import tpu as pltpu
from jax.experimental.pallas import tpu_sc as plsc

_NUM_CLASSES = 1000000
_EMBED_DIM = 64
_BATCH = 16384
_HIST = 50
_TOTAL = _BATCH * _HIST

_info = plsc.get_sparse_core_info()
_NC = _info.num_cores
_NS = _info.num_subcores
_NW = _NC * _NS
_B_PER_W = _TOTAL // _NW
_CHUNK = 128
_NCHUNKS = _B_PER_W // _CHUNK
_GROUP = 4
_GCHUNK = _GROUP * _CHUNK
_NGROUPS = _NCHUNKS // _GROUP

_mesh = plsc.VectorSubcoreMesh(core_axis_name="c", subcore_axis_name="s")


@functools.partial(
    pl.kernel,
    out_type=jax.ShapeDtypeStruct((_TOTAL, _EMBED_DIM), jnp.float32),
    mesh=_mesh,
    scratch_types=[
        pltpu.VMEM((_NCHUNKS, _CHUNK), jnp.int32),
        pltpu.VMEM((2, _GCHUNK, _EMBED_DIM), jnp.float32),
        pltpu.SemaphoreType.DMA,
        pltpu.SemaphoreType.DMA,
    ],
    compiler_params=pltpu.CompilerParams(use_tc_tiling_on_sc=False),
)
def _emb_lookup(idx_hbm, table_hbm, out_hbm, idx_v, rows_v, gsem, ssem):
    wid = lax.axis_index("s") * _NC + lax.axis_index("c")
    base = wid * _B_PER_W
    pltpu.sync_copy(idx_hbm.at[wid], idx_v)

    def fire_gathers(g, p):
        for b in range(_GROUP):
            pltpu.async_copy(
                table_hbm.at[idx_v.at[g * _GROUP + b]],
                rows_v.at[p, pl.ds(b * _CHUNK, _CHUNK)],
                gsem,
            )

    def wait_gathers(g, p):
        for b in range(_GROUP):
            pltpu.make_async_copy(
                table_hbm.at[idx_v.at[g * _GROUP + b]],
                rows_v.at[p, pl.ds(b * _CHUNK, _CHUNK)],
                gsem,
            ).wait()

    def fire_store(g, p):
        pltpu.async_copy(
            rows_v.at[p], out_hbm.at[pl.ds(base + g * _GCHUNK, _GCHUNK)], ssem
        )

    def wait_store(g, p):
        pltpu.make_async_copy(
            rows_v.at[p], out_hbm.at[pl.ds(base + g * _GCHUNK, _GCHUNK)], ssem
        ).wait()

    fire_gathers(0, 0)
    wait_gathers(0, 0)
    fire_store(0, 0)
    fire_gathers(1, 1)

    @pl.loop(0, (_NGROUPS - 2) // 2)
    def _steady(t):
        g = 2 * t + 1
        wait_gathers(g, 1)
        fire_store(g, 1)
        wait_store(g - 1, 0)
        fire_gathers(g + 1, 0)
        wait_gathers(g + 1, 0)
        fire_store(g + 1, 0)
        wait_store(g, 1)
        fire_gathers(g + 2, 1)

    g_last = _NGROUPS - 1
    wait_gathers(g_last, 1)
    fire_store(g_last, 1)
    wait_store(g_last - 1, 0)
    wait_store(g_last, 1)


def kernel(x, table):
    idx = x.reshape(_NW, _NCHUNKS, _CHUNK).astype(jnp.int32)
    out = _emb_lookup(idx, table)
    return out.reshape(_BATCH, _HIST, _EMBED_DIM)
